# baseline (speedup 1.0000x reference)
import jax
import jax.numpy as jnp
from jax.experimental import pallas as pl
from jax.experimental.pallas import tpu as pltpu


def _copy_row(lens_ref, in_ref, out_ref):
    del lens_ref
    out_ref[...] = in_ref[...]


def kernel(output, encoder_outputs, encoder_sequence_lengths):
    del output
    B, T, D = encoder_outputs.shape
    lengths = jnp.asarray(encoder_sequence_lengths, jnp.int32)
    flat = encoder_outputs.reshape(B * T, 1, D)

    grid_spec = pltpu.PrefetchScalarGridSpec(
        num_scalar_prefetch=1,
        grid=(B,),
        in_specs=[
            pl.BlockSpec((1, 1, D), lambda i, lens: (i * T + lens[i] - 1, 0, 0)),
        ],
        out_specs=pl.BlockSpec((1, 1, D), lambda i, lens: (i, 0, 0)),
    )
    out = pl.pallas_call(
        _copy_row,
        grid_spec=grid_spec,
        out_shape=jax.ShapeDtypeStruct((B, 1, D), jnp.float32),
    )(lengths, flat)
    return out.reshape(B, D)


# speedup vs baseline: 109.8856x; 109.8856x over previous
import jax
import jax.numpy as jnp
from jax.experimental import pallas as pl
from jax.experimental.pallas import tpu as pltpu


def _pick_row(lens_ref, in_ref, out_ref):
    i = pl.program_id(0)
    r = (lens_ref[i] - 1) % 8
    out_ref[pl.ds(i, 1), :] = in_ref[0, pl.ds(r, 1), :]


def kernel(output, encoder_outputs, encoder_sequence_lengths):
    del output
    B, T, D = encoder_outputs.shape
    lengths = jnp.asarray(encoder_sequence_lengths, jnp.int32)

    grid_spec = pltpu.PrefetchScalarGridSpec(
        num_scalar_prefetch=1,
        grid=(B,),
        in_specs=[
            pl.BlockSpec((1, 8, D), lambda i, lens: (i, (lens[i] - 1) // 8, 0)),
        ],
        out_specs=pl.BlockSpec((B, D), lambda i, lens: (0, 0)),
    )
    return pl.pallas_call(
        _pick_row,
        grid_spec=grid_spec,
        out_shape=jax.ShapeDtypeStruct((B, D), jnp.float32),
    )(lengths, encoder_outputs)
